# Initial kernel scaffold; baseline (speedup 1.0000x reference)
import jax
import jax.numpy as jnp
from jax.experimental import pallas as pl


def kernel(pred_logits, pred_masks, tgt_masks, tgt_labels):
    raise NotImplementedError("write your pallas kernel here")


# speedup vs baseline: 3.9232x; 3.9232x over previous
import numpy as np

import jax
import jax.numpy as jnp
from jax import lax
from jax.experimental import pallas as pl
from jax.experimental.pallas import tpu as pltpu

_B, _Q, _C = 2, 100, 81
_N = 20
_h = _w = 128
_H = _W = 512


def _interp_matrix(out_size: int, in_size: int) -> np.ndarray:
    o = np.arange(out_size, dtype=np.float32)
    src = (o + 0.5) * (in_size / out_size) - 0.5
    i0f = np.floor(src)
    frac = (src - i0f).astype(np.float32)
    i0 = np.clip(i0f.astype(np.int64), 0, in_size - 1)
    i1 = np.clip(i0f.astype(np.int64) + 1, 0, in_size - 1)
    A = np.zeros((out_size, in_size), dtype=np.float32)
    A[o.astype(np.int64), i0] += 1.0 - frac
    A[o.astype(np.int64), i1] += frac
    return A


_A_NP = _interp_matrix(_H, _h)


def _match_body(labels_ref, logits_ref, idx_ref, ss_ref):
    logits = logits_ref[0]
    mx = jnp.max(logits, axis=-1, keepdims=True)
    e = jnp.exp(logits - mx)
    prob = e / jnp.sum(e, axis=-1, keepdims=True)
    labels = labels_ref[0]
    iota_c = lax.broadcasted_iota(jnp.int32, (_C, _N), 0)
    onehot = (iota_c == labels).astype(jnp.float32)
    probT = lax.dot_general(onehot, prob, (((0,), (1,)), ((), ())),
                            precision=lax.Precision.HIGHEST,
                            preferred_element_type=jnp.float32)

    iota_row = lax.broadcasted_iota(jnp.int32, (_N, _Q), 0)
    iota_lane = lax.broadcasted_iota(jnp.int32, (1, _Q), 1)
    iota_tn = lax.broadcasted_iota(jnp.int32, (1, _N), 1)

    def step(t, carry):
        used, idxv, ssv = carry
        row = jnp.sum(jnp.where(iota_row == t, probT, 0.0), axis=0,
                      keepdims=True)
        c = jnp.where(used > 0.5, -jnp.inf, row)
        m = jnp.max(c)
        j = jnp.min(jnp.where(c == m, iota_lane, _Q))
        sel = iota_tn == t
        idxv = jnp.where(sel, j, idxv)
        ssv = jnp.where(sel, m, ssv)
        used = jnp.where(iota_lane == j, 1.0, used)
        return used, idxv, ssv

    used0 = jnp.zeros((1, _Q), dtype=jnp.float32)
    _, idxv, ssv = lax.fori_loop(
        0, _N, step,
        (used0, jnp.zeros((1, _N), jnp.int32), jnp.zeros((1, _N), jnp.float32)))
    idx_ref[0] = idxv
    ss_ref[0] = ssv


def _dense_body(idx_s, ss_s, pred_ref, tgt_ref, A_ref, out_ref, score_ref):
    b = pl.program_id(0)
    n = pl.program_id(1)
    x = pred_ref[0, 0]
    soft = 1.0 / (1.0 + jnp.exp(-x))
    hard = (soft > 0.5).astype(jnp.float32)
    num = jnp.sum(soft * hard)
    den = jnp.sum(hard)
    mask_score = num / (den + 1e-6)
    score = ss_s[b * _N + n] * mask_score

    A = A_ref[...]
    tmp = jnp.dot(A, soft, precision=lax.Precision.HIGHEST,
                  preferred_element_type=jnp.float32)
    up = lax.dot_general(tmp, A, (((1,), (1,)), ((), ())),
                         precision=lax.Precision.HIGHEST,
                         preferred_element_type=jnp.float32)
    out_ref[0, 0] = tgt_ref[0, 0] * (up > 0.5).astype(jnp.float32)
    score_ref[...] = jnp.full((1, 1, 1, 128), score, dtype=jnp.float32)


def kernel(pred_logits, pred_masks, tgt_masks, tgt_labels):
    B, Q, C = pred_logits.shape
    N = tgt_masks.shape[1]
    labels3 = tgt_labels.astype(jnp.int32).reshape(B, 1, N)

    idx, ss = pl.pallas_call(
        _match_body,
        grid=(B,),
        in_specs=[
            pl.BlockSpec((1, 1, N), lambda b: (b, 0, 0)),
            pl.BlockSpec((1, Q, C), lambda b: (b, 0, 0)),
        ],
        out_specs=[
            pl.BlockSpec((1, 1, N), lambda b: (b, 0, 0)),
            pl.BlockSpec((1, 1, N), lambda b: (b, 0, 0)),
        ],
        out_shape=[
            jax.ShapeDtypeStruct((B, 1, N), jnp.int32),
            jax.ShapeDtypeStruct((B, 1, N), jnp.float32),
        ],
    )(labels3, pred_logits)

    idx_flat = idx.reshape(B * N)
    ss_flat = ss.reshape(B * N)
    A = jnp.asarray(_A_NP)

    masks, scores_pad = pl.pallas_call(
        _dense_body,
        grid_spec=pltpu.PrefetchScalarGridSpec(
            num_scalar_prefetch=2,
            grid=(B, N),
            in_specs=[
                pl.BlockSpec((1, 1, _h, _w),
                             lambda b, n, idx_s, ss_s: (b, idx_s[b * N + n], 0, 0)),
                pl.BlockSpec((1, 1, _H, _W),
                             lambda b, n, idx_s, ss_s: (b, n, 0, 0)),
                pl.BlockSpec((_H, _h), lambda b, n, idx_s, ss_s: (0, 0)),
            ],
            out_specs=[
                pl.BlockSpec((1, 1, _H, _W),
                             lambda b, n, idx_s, ss_s: (b, n, 0, 0)),
                pl.BlockSpec((1, 1, 1, 128),
                             lambda b, n, idx_s, ss_s: (b, n, 0, 0)),
            ],
        ),
        out_shape=[
            jax.ShapeDtypeStruct((B, N, _H, _W), jnp.float32),
            jax.ShapeDtypeStruct((B, N, 1, 128), jnp.float32),
        ],
    )(idx_flat, ss_flat, pred_masks, tgt_masks, A)

    return scores_pad[:, :, 0, 0], masks


# speedup vs baseline: 3.9288x; 1.0014x over previous
import numpy as np

import jax
import jax.numpy as jnp
from jax import lax
from jax.experimental import pallas as pl
from jax.experimental.pallas import tpu as pltpu

_B, _Q, _C = 2, 100, 81
_N = 20
_h = _w = 128
_H = _W = 512


def _interp_matrix(out_size: int, in_size: int) -> np.ndarray:
    o = np.arange(out_size, dtype=np.float32)
    src = (o + 0.5) * (in_size / out_size) - 0.5
    i0f = np.floor(src)
    frac = (src - i0f).astype(np.float32)
    i0 = np.clip(i0f.astype(np.int64), 0, in_size - 1)
    i1 = np.clip(i0f.astype(np.int64) + 1, 0, in_size - 1)
    A = np.zeros((out_size, in_size), dtype=np.float32)
    A[o.astype(np.int64), i0] += 1.0 - frac
    A[o.astype(np.int64), i1] += frac
    return A


_A_NP = _interp_matrix(_H, _h)


def _match_body(labels_ref, logits_ref, idx_ref, ss_ref):
    logits = logits_ref[0]
    mx = jnp.max(logits, axis=-1, keepdims=True)
    e = jnp.exp(logits - mx)
    prob = e / jnp.sum(e, axis=-1, keepdims=True)
    labels = labels_ref[0]
    iota_c = lax.broadcasted_iota(jnp.int32, (_C, _N), 0)
    onehot = (iota_c == labels).astype(jnp.float32)
    probT = lax.dot_general(onehot, prob, (((0,), (1,)), ((), ())),
                            precision=lax.Precision.HIGHEST,
                            preferred_element_type=jnp.float32)

    iota_row = lax.broadcasted_iota(jnp.int32, (_N, _Q), 0)
    iota_lane = lax.broadcasted_iota(jnp.int32, (1, _Q), 1)
    iota_tn = lax.broadcasted_iota(jnp.int32, (1, _N), 1)

    def step(t, carry):
        used, idxv, ssv = carry
        row = jnp.sum(jnp.where(iota_row == t, probT, 0.0), axis=0,
                      keepdims=True)
        c = jnp.where(used > 0.5, -jnp.inf, row)
        m = jnp.max(c)
        j = jnp.min(jnp.where(c == m, iota_lane, _Q))
        sel = iota_tn == t
        idxv = jnp.where(sel, j, idxv)
        ssv = jnp.where(sel, m, ssv)
        used = jnp.where(iota_lane == j, 1.0, used)
        return used, idxv, ssv

    used0 = jnp.zeros((1, _Q), dtype=jnp.float32)
    _, idxv, ssv = lax.fori_loop(
        0, _N, step,
        (used0, jnp.zeros((1, _N), jnp.int32), jnp.zeros((1, _N), jnp.float32)))
    idx_ref[0] = idxv
    ss_ref[0] = ssv


def _dense_body(idx_s, ss_s, pred_ref, tgt_ref, A_ref, out_ref, score_ref):
    b = pl.program_id(0)
    n = pl.program_id(1)
    x = pred_ref[0, 0]
    soft = 1.0 / (1.0 + jnp.exp(-x))
    hard = (soft > 0.5).astype(jnp.float32)
    num = jnp.sum(soft * hard)
    den = jnp.sum(hard)
    mask_score = num / (den + 1e-6)
    score = ss_s[b * _N + n] * mask_score

    A = A_ref[...]
    tmp = jnp.dot(A, soft, precision=lax.Precision.HIGHEST,
                  preferred_element_type=jnp.float32)
    up = lax.dot_general(tmp, A, (((1,), (1,)), ((), ())),
                         precision=lax.Precision.HIGHEST,
                         preferred_element_type=jnp.float32)
    out_ref[0, 0] = tgt_ref[0, 0] * (up > 0.5).astype(jnp.float32)
    score_ref[...] = jnp.full((1, 1, 1, 128), score, dtype=jnp.float32)


def kernel(pred_logits, pred_masks, tgt_masks, tgt_labels):
    B, Q, C = pred_logits.shape
    N = tgt_masks.shape[1]
    labels3 = tgt_labels.astype(jnp.int32).reshape(B, 1, N)

    idx, ss = pl.pallas_call(
        _match_body,
        grid=(B,),
        in_specs=[
            pl.BlockSpec((1, 1, N), lambda b: (b, 0, 0)),
            pl.BlockSpec((1, Q, C), lambda b: (b, 0, 0)),
        ],
        out_specs=[
            pl.BlockSpec((1, 1, N), lambda b: (b, 0, 0)),
            pl.BlockSpec((1, 1, N), lambda b: (b, 0, 0)),
        ],
        out_shape=[
            jax.ShapeDtypeStruct((B, 1, N), jnp.int32),
            jax.ShapeDtypeStruct((B, 1, N), jnp.float32),
        ],
    )(labels3, pred_logits)

    idx_flat = idx.reshape(B * N)
    ss_flat = ss.reshape(B * N)
    A = jnp.asarray(_A_NP)

    masks, scores_pad = pl.pallas_call(
        _dense_body,
        grid_spec=pltpu.PrefetchScalarGridSpec(
            num_scalar_prefetch=2,
            grid=(B, N),
            in_specs=[
                pl.BlockSpec((1, 1, _h, _w),
                             lambda b, n, idx_s, ss_s: (b, idx_s[b * N + n], 0, 0)),
                pl.BlockSpec((1, 1, _H, _W),
                             lambda b, n, idx_s, ss_s: (b, n, 0, 0)),
                pl.BlockSpec((_H, _h), lambda b, n, idx_s, ss_s: (0, 0)),
            ],
            out_specs=[
                pl.BlockSpec((1, 1, _H, _W),
                             lambda b, n, idx_s, ss_s: (b, n, 0, 0)),
                pl.BlockSpec((1, 1, 1, 128),
                             lambda b, n, idx_s, ss_s: (b, n, 0, 0)),
            ],
        ),
        out_shape=[
            jax.ShapeDtypeStruct((B, N, _H, _W), jnp.float32),
            jax.ShapeDtypeStruct((B, N, 1, 128), jnp.float32),
        ],
        compiler_params=pltpu.CompilerParams(
            dimension_semantics=("parallel", "parallel")),
    )(idx_flat, ss_flat, pred_masks, tgt_masks, A)

    return scores_pad[:, :, 0, 0], masks


# speedup vs baseline: 4.0748x; 1.0372x over previous
import numpy as np

import jax
import jax.numpy as jnp
from jax import lax
from jax.experimental import pallas as pl
from jax.experimental.pallas import tpu as pltpu

_B, _Q, _C = 2, 100, 81
_N = 20
_h = _w = 128
_H = _W = 512
_NQCH = 4
_CH = _H // _NQCH
_KW = 64
_WSTART = tuple(min(max(32 * q - 8, 0), _h - _KW) for q in range(_NQCH))


def _interp_matrix(out_size: int, in_size: int) -> np.ndarray:
    o = np.arange(out_size, dtype=np.float32)
    src = (o + 0.5) * (in_size / out_size) - 0.5
    i0f = np.floor(src)
    frac = (src - i0f).astype(np.float32)
    i0 = np.clip(i0f.astype(np.int64), 0, in_size - 1)
    i1 = np.clip(i0f.astype(np.int64) + 1, 0, in_size - 1)
    A = np.zeros((out_size, in_size), dtype=np.float32)
    A[o.astype(np.int64), i0] += 1.0 - frac
    A[o.astype(np.int64), i1] += frac
    return A


_A_NP = _interp_matrix(_H, _h)
_ABAND_NP = np.stack(
    [_A_NP[q * _CH:(q + 1) * _CH, _WSTART[q]:_WSTART[q] + _KW]
     for q in range(_NQCH)], axis=0)
_ABAND_NP = _ABAND_NP.reshape(_NQCH * _CH, _KW)


def _match_body(labels_ref, logits_ref, idx_ref, ss_ref):
    logits = logits_ref[0]
    mx = jnp.max(logits, axis=-1, keepdims=True)
    e = jnp.exp(logits - mx)
    prob = e / jnp.sum(e, axis=-1, keepdims=True)
    labels = labels_ref[0]
    iota_c = lax.broadcasted_iota(jnp.int32, (_C, _N), 0)
    onehot = (iota_c == labels).astype(jnp.float32)
    probT = lax.dot_general(onehot, prob, (((0,), (1,)), ((), ())),
                            precision=lax.Precision.HIGHEST,
                            preferred_element_type=jnp.float32)

    iota_row = lax.broadcasted_iota(jnp.int32, (_N, _Q), 0)
    iota_lane = lax.broadcasted_iota(jnp.int32, (1, _Q), 1)
    iota_tn = lax.broadcasted_iota(jnp.int32, (1, _N), 1)

    def step(t, carry):
        used, idxv, ssv = carry
        row = jnp.sum(jnp.where(iota_row == t, probT, 0.0), axis=0,
                      keepdims=True)
        c = jnp.where(used > 0.5, -jnp.inf, row)
        m = jnp.max(c)
        j = jnp.min(jnp.where(c == m, iota_lane, _Q))
        sel = iota_tn == t
        idxv = jnp.where(sel, j, idxv)
        ssv = jnp.where(sel, m, ssv)
        used = jnp.where(iota_lane == j, 1.0, used)
        return used, idxv, ssv

    used0 = jnp.zeros((1, _Q), dtype=jnp.float32)
    _, idxv, ssv = lax.fori_loop(
        0, _N, step,
        (used0, jnp.zeros((1, _N), jnp.int32), jnp.zeros((1, _N), jnp.float32)))
    idx_ref[0] = idxv
    ss_ref[0] = ssv


def _dense_body(idx_s, ss_s, pred_ref, tgt_ref, At_ref, Ab_ref, out_ref,
                score_ref):
    b = pl.program_id(0)
    n = pl.program_id(1)
    x = pred_ref[0, 0]
    soft = 1.0 / (1.0 + jnp.exp(-x))
    hard = (soft > 0.5).astype(jnp.float32)
    num = jnp.sum(soft * hard)
    den = jnp.sum(hard)
    mask_score = num / (den + 1e-6)
    score = ss_s[b * _N + n] * mask_score

    At = At_ref[...]
    wide = jnp.dot(soft, At, precision=lax.Precision.HIGHEST,
                   preferred_element_type=jnp.float32)
    for q in range(_NQCH):
        aq = Ab_ref[q * _CH:(q + 1) * _CH, :]
        wq = wide[_WSTART[q]:_WSTART[q] + _KW, :]
        upq = jnp.dot(aq, wq, precision=lax.Precision.HIGHEST,
                      preferred_element_type=jnp.float32)
        out_ref[0, 0, q * _CH:(q + 1) * _CH, :] = (
            tgt_ref[0, 0, q * _CH:(q + 1) * _CH, :]
            * (upq > 0.5).astype(jnp.float32))
    score_ref[...] = jnp.full((1, 1, 1, 128), score, dtype=jnp.float32)


def kernel(pred_logits, pred_masks, tgt_masks, tgt_labels):
    B, Q, C = pred_logits.shape
    N = tgt_masks.shape[1]
    labels3 = tgt_labels.astype(jnp.int32).reshape(B, 1, N)

    idx, ss = pl.pallas_call(
        _match_body,
        grid=(B,),
        in_specs=[
            pl.BlockSpec((1, 1, N), lambda b: (b, 0, 0)),
            pl.BlockSpec((1, Q, C), lambda b: (b, 0, 0)),
        ],
        out_specs=[
            pl.BlockSpec((1, 1, N), lambda b: (b, 0, 0)),
            pl.BlockSpec((1, 1, N), lambda b: (b, 0, 0)),
        ],
        out_shape=[
            jax.ShapeDtypeStruct((B, 1, N), jnp.int32),
            jax.ShapeDtypeStruct((B, 1, N), jnp.float32),
        ],
    )(labels3, pred_logits)

    idx_flat = idx.reshape(B * N)
    ss_flat = ss.reshape(B * N)
    At = jnp.asarray(_A_NP.T.copy())
    Ab = jnp.asarray(_ABAND_NP)

    masks, scores_pad = pl.pallas_call(
        _dense_body,
        grid_spec=pltpu.PrefetchScalarGridSpec(
            num_scalar_prefetch=2,
            grid=(B, N),
            in_specs=[
                pl.BlockSpec((1, 1, _h, _w),
                             lambda b, n, idx_s, ss_s: (b, idx_s[b * N + n], 0, 0)),
                pl.BlockSpec((1, 1, _H, _W),
                             lambda b, n, idx_s, ss_s: (b, n, 0, 0)),
                pl.BlockSpec((_h, _W), lambda b, n, idx_s, ss_s: (0, 0)),
                pl.BlockSpec((_H, _KW), lambda b, n, idx_s, ss_s: (0, 0)),
            ],
            out_specs=[
                pl.BlockSpec((1, 1, _H, _W),
                             lambda b, n, idx_s, ss_s: (b, n, 0, 0)),
                pl.BlockSpec((1, 1, 1, 128),
                             lambda b, n, idx_s, ss_s: (b, n, 0, 0)),
            ],
        ),
        out_shape=[
            jax.ShapeDtypeStruct((B, N, _H, _W), jnp.float32),
            jax.ShapeDtypeStruct((B, N, 1, 128), jnp.float32),
        ],
    )(idx_flat, ss_flat, pred_masks, tgt_masks, At, Ab)

    return scores_pad[:, :, 0, 0], masks
